# baseline (speedup 1.0000x reference)
import functools

import jax
import jax.numpy as jnp
from jax import lax
from jax.experimental import pallas as pl
from jax.experimental.pallas import tpu as pltpu
from jax.experimental.pallas import tpu_sc as plsc

BATCH = 4096
EMBED_DIM = 64
NUM_WORKERS = 32
CHUNK = BATCH // NUM_WORKERS


def _sc_gather_fn(u_tab_hbm, m_tab_hbm, u_idx_hbm, m_idx_hbm,
                  ue_hbm, me_hbm,
                  u_idx_v, m_idx_v, u_rows_v, m_rows_v, sem_u, sem_m):
    wid = lax.axis_index("s") * 2 + lax.axis_index("c")
    base = wid * CHUNK
    pltpu.sync_copy(u_idx_hbm.at[pl.ds(base, CHUNK)], u_idx_v)
    pltpu.sync_copy(m_idx_hbm.at[pl.ds(base, CHUNK)], m_idx_v)
    cp_u = pltpu.async_copy(u_tab_hbm.at[u_idx_v], u_rows_v, sem_u)
    cp_m = pltpu.async_copy(m_tab_hbm.at[m_idx_v], m_rows_v, sem_m)
    cp_u.wait()
    cp_m.wait()
    pltpu.sync_copy(u_rows_v, ue_hbm.at[pl.ds(base, CHUNK)])
    pltpu.sync_copy(m_rows_v, me_hbm.at[pl.ds(base, CHUNK)])


def _sc_gather(users, movies, user_table, movie_table):
    mesh = plsc.VectorSubcoreMesh(core_axis_name="c", subcore_axis_name="s")
    row = jax.ShapeDtypeStruct((BATCH, EMBED_DIM), jnp.float32)
    k = pl.kernel(
        _sc_gather_fn,
        out_type=(row, row),
        mesh=mesh,
        compiler_params=pltpu.CompilerParams(use_tc_tiling_on_sc=False),
        scratch_types=[
            pltpu.VMEM((CHUNK,), jnp.int32),
            pltpu.VMEM((CHUNK,), jnp.int32),
            pltpu.VMEM((CHUNK, EMBED_DIM), jnp.float32),
            pltpu.VMEM((CHUNK, EMBED_DIM), jnp.float32),
            pltpu.SemaphoreType.DMA,
            pltpu.SemaphoreType.DMA,
        ],
    )
    return k(user_table, movie_table, users, movies)


def _bn(x, g, be, eps=1e-5):
    mu = jnp.mean(x, axis=0, keepdims=True)
    var = jnp.mean((x - mu) ** 2, axis=0, keepdims=True)
    return (x - mu) * (g * lax.rsqrt(var + eps)) + be


def _mlp_fn(ue, me, w1u, w1m, b1, g1, be1, w2, b2, g2, be2,
            w3, b3, g3, be3, wp, bp, o_ref):
    x = jnp.dot(ue[...], w1u[...], preferred_element_type=jnp.float32)
    x = x + jnp.dot(me[...], w1m[...], preferred_element_type=jnp.float32)
    x = jnp.maximum(x + b1[...], 0.0)
    x = _bn(x, g1[...], be1[...])
    x = jnp.dot(x, w2[...], preferred_element_type=jnp.float32)
    x = jnp.maximum(x + b2[...], 0.0)
    x = _bn(x, g2[...], be2[...])
    x = jnp.dot(x, w3[...], preferred_element_type=jnp.float32)
    x = jnp.maximum(x + b3[...], 0.0)
    x = _bn(x, g3[...], be3[...])
    p = jnp.sum(x * wp[...], axis=1, keepdims=True) + bp[...]
    o_ref[...] = jax.nn.sigmoid(p) * 5.0


def _tc_mlp(ue, me, W1, b1, g1, be1, W2, b2, g2, be2, W3, b3, g3, be3, Wp, bp):
    return pl.pallas_call(
        _mlp_fn,
        out_shape=jax.ShapeDtypeStruct((BATCH, 1), jnp.float32),
    )(ue, me,
      W1[:EMBED_DIM], W1[EMBED_DIM:],
      b1.reshape(1, -1), g1.reshape(1, -1), be1.reshape(1, -1),
      W2, b2.reshape(1, -1), g2.reshape(1, -1), be2.reshape(1, -1),
      W3, b3.reshape(1, -1), g3.reshape(1, -1), be3.reshape(1, -1),
      Wp.reshape(1, -1), bp.reshape(1, 1))


def kernel(users, movies, user_table, movie_table,
           W1, b1, g1, be1, W2, b2, g2, be2, W3, b3, g3, be3, Wp, bp):
    ue, me = _sc_gather(users.astype(jnp.int32), movies.astype(jnp.int32),
                        user_table, movie_table)
    return _tc_mlp(ue, me, W1, b1, g1, be1, W2, b2, g2, be2,
                   W3, b3, g3, be3, Wp, bp)
